# baseline (speedup 1.0000x reference)
import dataclasses
import functools

import jax
import jax.numpy as jnp
from jax import lax
from jax.experimental import pallas as pl
from jax.experimental.pallas import tpu as pltpu
from jax.experimental.pallas import tpu_sc as plsc

N_NODES = 10000
N_EDGES = 320000
D = 128
N_GRAPHS = 64

NC, NS = 2, 16
NW = NC * NS
CHUNK = 128
N_CH = 80
NBUF = 2
E_PAD = NW * N_CH * CHUNK
NPAD = 10240
RPT = NPAD // NS
RBLK = 1024
N_RBLK = NPAD // RBLK

_mesh = plsc.VectorSubcoreMesh(core_axis_name="c", subcore_axis_name="s")

_cp = pltpu.CompilerParams()
if "needs_layout_passes" in pltpu.CompilerParams.__dataclass_fields__:
    _cp = dataclasses.replace(_cp, needs_layout_passes=False)


@functools.partial(
    pl.kernel,
    mesh=_mesh,
    compiler_params=_cp,
    out_type=jax.ShapeDtypeStruct((NW, NPAD), jnp.float32),
    scratch_types=[
        pltpu.VMEM((N_CH, CHUNK), jnp.int32),
        pltpu.VMEM((NPAD,), jnp.float32),
    ],
)
def _deg_hist(dst_hbm, zeros_hbm, out_hbm, idx_v, hist_v):
    cid = lax.axis_index("c")
    sid = lax.axis_index("s")
    wid = cid * NS + sid
    pltpu.sync_copy(zeros_hbm, hist_v)
    pltpu.sync_copy(dst_hbm.at[wid], idx_v)
    ones = jnp.ones((16,), jnp.float32)

    @pl.loop(0, N_CH)
    def _(j):
        for c in range(CHUNK // 16):
            idx16 = idx_v[j, pl.ds(c * 16, 16)]
            plsc.addupdate_scatter(hist_v, [idx16], ones)

    pltpu.sync_copy(hist_v, out_hbm.at[wid])


@functools.partial(
    pl.kernel,
    mesh=_mesh,
    out_type=jax.ShapeDtypeStruct((NC, NPAD, D), jnp.float32),
    scratch_types=[
        pltpu.VMEM((N_CH, CHUNK), jnp.int32),
        pltpu.VMEM((N_CH, CHUNK), jnp.int32),
        pltpu.VMEM((CHUNK, D), jnp.float32),
        pltpu.VMEM_SHARED((NPAD, D), jnp.float32),
        pltpu.SemaphoreType.DMA,
    ],
)
def _edge_scatter(y_hbm, src_hbm, dst_hbm, zrows_hbm, out_hbm,
                  src_v, dst_v, rows_v, acc_sh, sem):
    cid = lax.axis_index("c")
    sid = lax.axis_index("s")
    wid = cid * NS + sid
    pltpu.sync_copy(zrows_hbm, acc_sh.at[pl.ds(sid * RPT, RPT)])
    pltpu.sync_copy(src_hbm.at[wid], src_v)
    pltpu.sync_copy(dst_hbm.at[wid], dst_v)
    plsc.subcore_barrier()

    @pl.loop(0, N_CH)
    def _(j):
        pltpu.async_copy(y_hbm.at[src_v.at[j]], rows_v, sem).wait()
        pltpu.sync_copy(rows_v, acc_sh.at[dst_v.at[j]], add=True)

    plsc.subcore_barrier()
    pltpu.sync_copy(acc_sh.at[pl.ds(sid * RPT, RPT)],
                    out_hbm.at[cid, pl.ds(sid * RPT, RPT)])


def _mm_body(x_ref, w_ref, o_ref):
    o_ref[...] = jnp.dot(x_ref[...], w_ref[...],
                         preferred_element_type=jnp.float32,
                         precision=lax.Precision.HIGHEST)


_mm = pl.pallas_call(
    _mm_body,
    grid=(N_RBLK,),
    in_specs=[pl.BlockSpec((RBLK, D), lambda i: (i, 0)),
              pl.BlockSpec((D, D), lambda i: (0, 0))],
    out_specs=pl.BlockSpec((RBLK, D), lambda i: (i, 0)),
    out_shape=jax.ShapeDtypeStruct((NPAD, D), jnp.float32),
)


def _dinv_of(deg_ref):
    deg = jnp.sum(deg_ref[...], axis=0) + 1.0
    return lax.rsqrt(deg)[:, None]


def _scale_body(deg_ref, v_ref, o_ref):
    o_ref[...] = _dinv_of(deg_ref) * v_ref[...]


_scale = pl.pallas_call(
    _scale_body,
    grid=(N_RBLK,),
    in_specs=[pl.BlockSpec((NW, RBLK), lambda i: (0, i)),
              pl.BlockSpec((RBLK, D), lambda i: (i, 0))],
    out_specs=pl.BlockSpec((RBLK, D), lambda i: (i, 0)),
    out_shape=jax.ShapeDtypeStruct((NPAD, D), jnp.float32),
)


def _layer_body(deg_ref, s_ref, y_ref, b_ref, w_ref, o_ref):
    dinv = _dinv_of(deg_ref)
    h = dinv * (s_ref[0] + s_ref[1] + y_ref[...]) + b_ref[...]
    h = jnp.maximum(h, 0.0)
    o_ref[...] = dinv * jnp.dot(h, w_ref[...],
                                preferred_element_type=jnp.float32,
                                precision=lax.Precision.HIGHEST)


_layer = pl.pallas_call(
    _layer_body,
    grid=(N_RBLK,),
    in_specs=[pl.BlockSpec((NW, RBLK), lambda i: (0, i)),
              pl.BlockSpec((NC, RBLK, D), lambda i: (0, i, 0)),
              pl.BlockSpec((RBLK, D), lambda i: (i, 0)),
              pl.BlockSpec((1, D), lambda i: (0, 0)),
              pl.BlockSpec((D, D), lambda i: (0, 0))],
    out_specs=pl.BlockSpec((RBLK, D), lambda i: (i, 0)),
    out_shape=jax.ShapeDtypeStruct((NPAD, D), jnp.float32),
)


def _pool_body(deg_ref, t_ref, z_ref, batch_ref, b2_ref, wp_ref, bp_ref,
               emb_ref, out_ref, acc, cnt):
    i = pl.program_id(0)

    @pl.when(i == 0)
    def _():
        acc[...] = jnp.zeros_like(acc)
        cnt[...] = jnp.zeros_like(cnt)

    dinv = _dinv_of(deg_ref)
    h2 = dinv * (t_ref[0] + t_ref[1] + z_ref[...])
    b = batch_ref[0, 0, :]
    gids = lax.broadcasted_iota(jnp.int32, (N_GRAPHS, RBLK), 0)
    oh = (b[None, :] == gids).astype(jnp.float32)
    acc[...] += jnp.dot(oh, h2, preferred_element_type=jnp.float32,
                        precision=lax.Precision.HIGHEST)
    csum = jnp.sum(oh, axis=1, keepdims=True)
    cnt[...] += jnp.broadcast_to(csum, (N_GRAPHS, D))

    @pl.when(i == N_RBLK - 1)
    def _():
        emb = acc[...] / jnp.maximum(cnt[...], 1.0) + b2_ref[...]
        emb_ref[...] = emb
        out_ref[...] = jnp.dot(emb, wp_ref[...],
                               preferred_element_type=jnp.float32,
                               precision=lax.Precision.HIGHEST) + bp_ref[...]


_pool = pl.pallas_call(
    _pool_body,
    grid=(N_RBLK,),
    in_specs=[pl.BlockSpec((NW, RBLK), lambda i: (0, i)),
              pl.BlockSpec((NC, RBLK, D), lambda i: (0, i, 0)),
              pl.BlockSpec((RBLK, D), lambda i: (i, 0)),
              pl.BlockSpec((1, 1, RBLK), lambda i: (i, 0, 0)),
              pl.BlockSpec((1, D), lambda i: (0, 0)),
              pl.BlockSpec((D, D), lambda i: (0, 0)),
              pl.BlockSpec((1, D), lambda i: (0, 0))],
    out_specs=[pl.BlockSpec((N_GRAPHS, D), lambda i: (0, 0)),
               pl.BlockSpec((N_GRAPHS, D), lambda i: (0, 0))],
    out_shape=[jax.ShapeDtypeStruct((N_GRAPHS, D), jnp.float32),
               jax.ShapeDtypeStruct((N_GRAPHS, D), jnp.float32)],
    scratch_shapes=[pltpu.VMEM((N_GRAPHS, D), jnp.float32),
                    pltpu.VMEM((N_GRAPHS, D), jnp.float32)],
)


def kernel(x, edge_index, batch, W1, b1, W2, b2, Wp, bp):
    src = edge_index[0].astype(jnp.int32)
    dst = edge_index[1].astype(jnp.int32)
    epad = E_PAD - N_EDGES
    dummy = jnp.full((epad,), N_NODES, dtype=jnp.int32)
    src_t = jnp.concatenate([src, dummy]).reshape(NW, N_CH, CHUNK)
    dst_t = jnp.concatenate([dst, dummy]).reshape(NW, N_CH, CHUNK)

    x_pad = jnp.zeros((NPAD, D), jnp.float32).at[:N_NODES].set(x)
    batch_pad = jnp.full((NPAD,), N_GRAPHS, jnp.int32).at[:N_NODES].set(
        batch.astype(jnp.int32)).reshape(N_RBLK, 1, RBLK)

    zeros_n = jnp.zeros((NPAD,), jnp.float32)
    zrows = jnp.zeros((RPT, D), jnp.float32)
    b1r = b1.reshape(1, D)
    b2r = b2.reshape(1, D)
    wp_pad = jnp.zeros((D, D), jnp.float32).at[:, :1].set(Wp)
    bp_pad = jnp.zeros((1, D), jnp.float32).at[0, 0].set(bp[0])

    deg_parts = _deg_hist(dst_t, zeros_n)
    v1 = _mm(x_pad, W1)
    y = _scale(deg_parts, v1)
    s_parts = _edge_scatter(y, src_t, dst_t, zrows)
    z = _layer(deg_parts, s_parts, y, b1r, W2)
    t_parts = _edge_scatter(z, src_t, dst_t, zrows)
    emb, out_full = _pool(deg_parts, t_parts, z, batch_pad, b2r, wp_pad, bp_pad)
    return (out_full[:, :1], emb)


# speedup vs baseline: 3.8014x; 3.8014x over previous
import dataclasses
import functools

import jax
import jax.numpy as jnp
from jax import lax
from jax.experimental import pallas as pl
from jax.experimental.pallas import tpu as pltpu
from jax.experimental.pallas import tpu_sc as plsc

N_NODES = 10000
N_EDGES = 320000
D = 128
N_GRAPHS = 64

NC, NS = 2, 16
NW = NC * NS
CHUNK = 128
N_CH = 80
NBUF = 2
E_PAD = NW * N_CH * CHUNK
NPAD = 10240
RPT = NPAD // NS
RBLK = 1024
N_RBLK = NPAD // RBLK

_mesh = plsc.VectorSubcoreMesh(core_axis_name="c", subcore_axis_name="s")

_cp = pltpu.CompilerParams()
if "needs_layout_passes" in pltpu.CompilerParams.__dataclass_fields__:
    _cp = dataclasses.replace(_cp, needs_layout_passes=False)


@functools.partial(
    pl.kernel,
    mesh=_mesh,
    compiler_params=_cp,
    out_type=jax.ShapeDtypeStruct((NW, NPAD), jnp.float32),
    scratch_types=[
        pltpu.VMEM((N_CH, CHUNK), jnp.int32),
        pltpu.VMEM((NPAD,), jnp.float32),
    ],
)
def _deg_hist(dst_hbm, zeros_hbm, out_hbm, idx_v, hist_v):
    cid = lax.axis_index("c")
    sid = lax.axis_index("s")
    wid = cid * NS + sid
    pltpu.sync_copy(zeros_hbm, hist_v)
    pltpu.sync_copy(dst_hbm.at[wid], idx_v)
    ones = jnp.ones((16,), jnp.float32)

    @pl.loop(0, N_CH)
    def _(j):
        for c in range(CHUNK // 16):
            idx16 = idx_v[j, pl.ds(c * 16, 16)]
            plsc.addupdate_scatter(hist_v, [idx16], ones)

    pltpu.sync_copy(hist_v, out_hbm.at[wid])


PH = 2
PCH = N_CH // PH


@functools.partial(
    pl.kernel,
    mesh=_mesh,
    out_type=jax.ShapeDtypeStruct((NC, NPAD, D), jnp.float32),
    scratch_types=[
        pltpu.VMEM((PCH, CHUNK), jnp.int32),
        pltpu.VMEM((PCH, CHUNK), jnp.int32),
    ] + [pltpu.VMEM((CHUNK, D), jnp.float32)] * NBUF
      + [pltpu.VMEM_SHARED((NPAD, D), jnp.float32)]
      + [pltpu.SemaphoreType.DMA] * NBUF,
)
def _edge_scatter(y_hbm, src_hbm, dst_hbm, zrows_hbm, out_hbm,
                  src_v, dst_v, r0, r1, acc_sh, g0, g1):
    rows = (r0, r1)
    gsem = (g0, g1)
    cid = lax.axis_index("c")
    sid = lax.axis_index("s")
    wid = cid * NS + sid
    pltpu.sync_copy(zrows_hbm, acc_sh.at[pl.ds(sid * RPT, RPT)])
    plsc.subcore_barrier()

    for p in range(PH):
        pltpu.sync_copy(src_hbm.at[wid, pl.ds(p * PCH, PCH)], src_v)
        pltpu.sync_copy(dst_hbm.at[wid, pl.ds(p * PCH, PCH)], dst_v)
        for b in range(NBUF):
            pltpu.async_copy(y_hbm.at[src_v.at[b]], rows[b], gsem[b])

        @pl.loop(0, PCH // NBUF)
        def _(o):
            for b in range(NBUF):
                k = o * NBUF + b
                pltpu.make_async_copy(
                    y_hbm.at[src_v.at[k]], rows[b], gsem[b]).wait()
                pltpu.sync_copy(rows[b], acc_sh.at[dst_v.at[k]], add=True)

                @pl.when(k + NBUF < PCH)
                def _():
                    pltpu.async_copy(
                        y_hbm.at[src_v.at[k + NBUF]], rows[b], gsem[b])

    plsc.subcore_barrier()
    pltpu.sync_copy(acc_sh.at[pl.ds(sid * RPT, RPT)],
                    out_hbm.at[cid, pl.ds(sid * RPT, RPT)])


def _mm_body(x_ref, w_ref, o_ref):
    o_ref[...] = jnp.dot(x_ref[...], w_ref[...],
                         preferred_element_type=jnp.float32,
                         precision=lax.Precision.HIGHEST)


_mm = pl.pallas_call(
    _mm_body,
    grid=(N_RBLK,),
    in_specs=[pl.BlockSpec((RBLK, D), lambda i: (i, 0)),
              pl.BlockSpec((D, D), lambda i: (0, 0))],
    out_specs=pl.BlockSpec((RBLK, D), lambda i: (i, 0)),
    out_shape=jax.ShapeDtypeStruct((NPAD, D), jnp.float32),
)


def _dinv_of(deg_ref):
    deg = jnp.sum(deg_ref[...], axis=0) + 1.0
    return lax.rsqrt(deg)[:, None]


def _scale_body(deg_ref, v_ref, o_ref):
    o_ref[...] = _dinv_of(deg_ref) * v_ref[...]


_scale = pl.pallas_call(
    _scale_body,
    grid=(N_RBLK,),
    in_specs=[pl.BlockSpec((NW, RBLK), lambda i: (0, i)),
              pl.BlockSpec((RBLK, D), lambda i: (i, 0))],
    out_specs=pl.BlockSpec((RBLK, D), lambda i: (i, 0)),
    out_shape=jax.ShapeDtypeStruct((NPAD, D), jnp.float32),
)


def _layer_body(deg_ref, s_ref, y_ref, b_ref, w_ref, o_ref):
    dinv = _dinv_of(deg_ref)
    h = dinv * (s_ref[0] + s_ref[1] + y_ref[...]) + b_ref[...]
    h = jnp.maximum(h, 0.0)
    o_ref[...] = dinv * jnp.dot(h, w_ref[...],
                                preferred_element_type=jnp.float32,
                                precision=lax.Precision.HIGHEST)


_layer = pl.pallas_call(
    _layer_body,
    grid=(N_RBLK,),
    in_specs=[pl.BlockSpec((NW, RBLK), lambda i: (0, i)),
              pl.BlockSpec((NC, RBLK, D), lambda i: (0, i, 0)),
              pl.BlockSpec((RBLK, D), lambda i: (i, 0)),
              pl.BlockSpec((1, D), lambda i: (0, 0)),
              pl.BlockSpec((D, D), lambda i: (0, 0))],
    out_specs=pl.BlockSpec((RBLK, D), lambda i: (i, 0)),
    out_shape=jax.ShapeDtypeStruct((NPAD, D), jnp.float32),
)


def _pool_body(deg_ref, t_ref, z_ref, batch_ref, b2_ref, wp_ref, bp_ref,
               emb_ref, out_ref, acc, cnt):
    i = pl.program_id(0)

    @pl.when(i == 0)
    def _():
        acc[...] = jnp.zeros_like(acc)
        cnt[...] = jnp.zeros_like(cnt)

    dinv = _dinv_of(deg_ref)
    h2 = dinv * (t_ref[0] + t_ref[1] + z_ref[...])
    b = batch_ref[0, 0, :]
    gids = lax.broadcasted_iota(jnp.int32, (N_GRAPHS, RBLK), 0)
    oh = (b[None, :] == gids).astype(jnp.float32)
    acc[...] += jnp.dot(oh, h2, preferred_element_type=jnp.float32,
                        precision=lax.Precision.HIGHEST)
    csum = jnp.sum(oh, axis=1, keepdims=True)
    cnt[...] += jnp.broadcast_to(csum, (N_GRAPHS, D))

    @pl.when(i == N_RBLK - 1)
    def _():
        emb = acc[...] / jnp.maximum(cnt[...], 1.0) + b2_ref[...]
        emb_ref[...] = emb
        out_ref[...] = jnp.dot(emb, wp_ref[...],
                               preferred_element_type=jnp.float32,
                               precision=lax.Precision.HIGHEST) + bp_ref[...]


_pool = pl.pallas_call(
    _pool_body,
    grid=(N_RBLK,),
    in_specs=[pl.BlockSpec((NW, RBLK), lambda i: (0, i)),
              pl.BlockSpec((NC, RBLK, D), lambda i: (0, i, 0)),
              pl.BlockSpec((RBLK, D), lambda i: (i, 0)),
              pl.BlockSpec((1, 1, RBLK), lambda i: (i, 0, 0)),
              pl.BlockSpec((1, D), lambda i: (0, 0)),
              pl.BlockSpec((D, D), lambda i: (0, 0)),
              pl.BlockSpec((1, D), lambda i: (0, 0))],
    out_specs=[pl.BlockSpec((N_GRAPHS, D), lambda i: (0, 0)),
               pl.BlockSpec((N_GRAPHS, D), lambda i: (0, 0))],
    out_shape=[jax.ShapeDtypeStruct((N_GRAPHS, D), jnp.float32),
               jax.ShapeDtypeStruct((N_GRAPHS, D), jnp.float32)],
    scratch_shapes=[pltpu.VMEM((N_GRAPHS, D), jnp.float32),
                    pltpu.VMEM((N_GRAPHS, D), jnp.float32)],
)


def kernel(x, edge_index, batch, W1, b1, W2, b2, Wp, bp):
    src = edge_index[0].astype(jnp.int32)
    dst = edge_index[1].astype(jnp.int32)
    epad = E_PAD - N_EDGES
    dummy = N_NODES + (jnp.arange(epad, dtype=jnp.int32) % (NPAD - N_NODES))
    src_t = jnp.concatenate([src, dummy]).reshape(NW, N_CH, CHUNK)
    dst_t = jnp.concatenate([dst, dummy]).reshape(NW, N_CH, CHUNK)

    x_pad = jnp.zeros((NPAD, D), jnp.float32).at[:N_NODES].set(x)
    batch_pad = jnp.full((NPAD,), N_GRAPHS, jnp.int32).at[:N_NODES].set(
        batch.astype(jnp.int32)).reshape(N_RBLK, 1, RBLK)

    zeros_n = jnp.zeros((NPAD,), jnp.float32)
    zrows = jnp.zeros((RPT, D), jnp.float32)
    b1r = b1.reshape(1, D)
    b2r = b2.reshape(1, D)
    wp_pad = jnp.zeros((D, D), jnp.float32).at[:, :1].set(Wp)
    bp_pad = jnp.zeros((1, D), jnp.float32).at[0, 0].set(bp[0])

    deg_parts = _deg_hist(dst_t, zeros_n)
    v1 = _mm(x_pad, W1)
    y = _scale(deg_parts, v1)
    s_parts = _edge_scatter(y, src_t, dst_t, zrows)
    z = _layer(deg_parts, s_parts, y, b1r, W2)
    t_parts = _edge_scatter(z, src_t, dst_t, zrows)
    emb, out_full = _pool(deg_parts, t_parts, z, batch_pad, b2r, wp_pad, bp_pad)
    return (out_full[:, :1], emb)


# speedup vs baseline: 3.8484x; 1.0124x over previous
import dataclasses
import functools

import jax
import jax.numpy as jnp
from jax import lax
from jax.experimental import pallas as pl
from jax.experimental.pallas import tpu as pltpu
from jax.experimental.pallas import tpu_sc as plsc

N_NODES = 10000
N_EDGES = 320000
D = 128
N_GRAPHS = 64

NC, NS = 2, 16
NW = NC * NS
CHUNK = 128
N_CH = 80
NBUF = 2
E_PAD = NW * N_CH * CHUNK
NPAD = 10240
RPT = NPAD // NS
RBLK = 1024
N_RBLK = NPAD // RBLK

_mesh = plsc.VectorSubcoreMesh(core_axis_name="c", subcore_axis_name="s")

_cp = pltpu.CompilerParams()
if "needs_layout_passes" in pltpu.CompilerParams.__dataclass_fields__:
    _cp = dataclasses.replace(_cp, needs_layout_passes=False)


@functools.partial(
    pl.kernel,
    mesh=_mesh,
    compiler_params=_cp,
    out_type=jax.ShapeDtypeStruct((NW, NPAD), jnp.float32),
    scratch_types=[
        pltpu.VMEM((N_CH, CHUNK), jnp.int32),
        pltpu.VMEM((NPAD,), jnp.float32),
    ],
)
def _deg_hist(dst_hbm, zeros_hbm, out_hbm, idx_v, hist_v):
    cid = lax.axis_index("c")
    sid = lax.axis_index("s")
    wid = cid * NS + sid
    pltpu.sync_copy(zeros_hbm, hist_v)
    pltpu.sync_copy(dst_hbm.at[wid], idx_v)
    ones = jnp.ones((16,), jnp.float32)

    @pl.loop(0, N_CH)
    def _(j):
        for c in range(CHUNK // 16):
            idx16 = idx_v[j, pl.ds(c * 16, 16)]
            plsc.addupdate_scatter(hist_v, [idx16], ones)

    pltpu.sync_copy(hist_v, out_hbm.at[wid])


PH = 2
PCH = N_CH // PH


@functools.partial(
    pl.kernel,
    mesh=_mesh,
    out_type=jax.ShapeDtypeStruct((NC, NPAD, D), jnp.float32),
    scratch_types=[
        pltpu.VMEM((PCH, CHUNK), jnp.int32),
        pltpu.VMEM((PCH, CHUNK), jnp.int32),
    ] + [pltpu.VMEM((CHUNK, D), jnp.float32)] * NBUF
      + [pltpu.VMEM_SHARED((NPAD, D), jnp.float32)]
      + [pltpu.SemaphoreType.DMA] * NBUF,
)
def _edge_scatter(y_hbm, src_hbm, dst_hbm, zrows_hbm, out_hbm,
                  src_v, dst_v, r0, r1, acc_sh, g0, g1):
    rows = (r0, r1)
    gsem = (g0, g1)
    cid = lax.axis_index("c")
    sid = lax.axis_index("s")
    wid = cid * NS + sid
    pltpu.sync_copy(zrows_hbm, acc_sh.at[pl.ds(sid * RPT, RPT)])
    plsc.subcore_barrier()

    for p in range(PH):
        pltpu.sync_copy(src_hbm.at[wid, pl.ds(p * PCH, PCH)], src_v)
        pltpu.sync_copy(dst_hbm.at[wid, pl.ds(p * PCH, PCH)], dst_v)
        for b in range(NBUF):
            pltpu.async_copy(y_hbm.at[src_v.at[b]], rows[b], gsem[b])

        @pl.loop(0, PCH // NBUF)
        def _(o):
            for b in range(NBUF):
                k = o * NBUF + b
                pltpu.make_async_copy(
                    y_hbm.at[src_v.at[k]], rows[b], gsem[b]).wait()
                pltpu.sync_copy(rows[b], acc_sh.at[dst_v.at[k]], add=True)

                @pl.when(k + NBUF < PCH)
                def _():
                    pltpu.async_copy(
                        y_hbm.at[src_v.at[k + NBUF]], rows[b], gsem[b])

    plsc.subcore_barrier()
    pltpu.sync_copy(acc_sh.at[pl.ds(sid * RPT, RPT)],
                    out_hbm.at[cid, pl.ds(sid * RPT, RPT)])


def _mm_body(x_ref, w_ref, o_ref):
    o_ref[...] = jnp.dot(x_ref[...].astype(jnp.bfloat16),
                         w_ref[...].astype(jnp.bfloat16),
                         preferred_element_type=jnp.float32)


_mm = pl.pallas_call(
    _mm_body,
    grid=(N_RBLK,),
    in_specs=[pl.BlockSpec((RBLK, D), lambda i: (i, 0)),
              pl.BlockSpec((D, D), lambda i: (0, 0))],
    out_specs=pl.BlockSpec((RBLK, D), lambda i: (i, 0)),
    out_shape=jax.ShapeDtypeStruct((NPAD, D), jnp.float32),
)


def _dinv_of(deg_ref):
    deg = jnp.sum(deg_ref[...], axis=0) + 1.0
    return (1.0 / jnp.sqrt(deg))[:, None]


def _scale_body(deg_ref, v_ref, o_ref):
    o_ref[...] = _dinv_of(deg_ref) * v_ref[...]


_scale = pl.pallas_call(
    _scale_body,
    grid=(N_RBLK,),
    in_specs=[pl.BlockSpec((NW, RBLK), lambda i: (0, i)),
              pl.BlockSpec((RBLK, D), lambda i: (i, 0))],
    out_specs=pl.BlockSpec((RBLK, D), lambda i: (i, 0)),
    out_shape=jax.ShapeDtypeStruct((NPAD, D), jnp.float32),
)


def _layer_body(deg_ref, s_ref, y_ref, b_ref, w_ref, o_ref):
    dinv = _dinv_of(deg_ref)
    h = dinv * (s_ref[0] + s_ref[1] + y_ref[...]) + b_ref[...]
    h = jnp.maximum(h, 0.0)
    o_ref[...] = dinv * jnp.dot(h.astype(jnp.bfloat16),
                                w_ref[...].astype(jnp.bfloat16),
                                preferred_element_type=jnp.float32)


_layer = pl.pallas_call(
    _layer_body,
    grid=(N_RBLK,),
    in_specs=[pl.BlockSpec((NW, RBLK), lambda i: (0, i)),
              pl.BlockSpec((NC, RBLK, D), lambda i: (0, i, 0)),
              pl.BlockSpec((RBLK, D), lambda i: (i, 0)),
              pl.BlockSpec((1, D), lambda i: (0, 0)),
              pl.BlockSpec((D, D), lambda i: (0, 0))],
    out_specs=pl.BlockSpec((RBLK, D), lambda i: (i, 0)),
    out_shape=jax.ShapeDtypeStruct((NPAD, D), jnp.float32),
)


def _pool_body(deg_ref, t_ref, z_ref, batch_ref, b2_ref, wp_ref, bp_ref,
               emb_ref, out_ref, acc, cnt):
    i = pl.program_id(0)

    @pl.when(i == 0)
    def _():
        acc[...] = jnp.zeros_like(acc)
        cnt[...] = jnp.zeros_like(cnt)

    dinv = _dinv_of(deg_ref)
    h2 = dinv * (t_ref[0] + t_ref[1] + z_ref[...])
    b = batch_ref[0, 0, :]
    gids = lax.broadcasted_iota(jnp.int32, (N_GRAPHS, RBLK), 0)
    oh = (b[None, :] == gids).astype(jnp.float32)
    acc[...] += jnp.dot(oh, h2, preferred_element_type=jnp.float32,
                        precision=lax.Precision.HIGHEST)
    csum = jnp.sum(oh, axis=1, keepdims=True)
    cnt[...] += jnp.broadcast_to(csum, (N_GRAPHS, D))

    @pl.when(i == N_RBLK - 1)
    def _():
        emb = acc[...] / jnp.maximum(cnt[...], 1.0) + b2_ref[...]
        emb_ref[...] = emb
        out_ref[...] = jnp.dot(emb.astype(jnp.bfloat16),
                               wp_ref[...].astype(jnp.bfloat16),
                               preferred_element_type=jnp.float32) + bp_ref[...]


_pool = pl.pallas_call(
    _pool_body,
    grid=(N_RBLK,),
    in_specs=[pl.BlockSpec((NW, RBLK), lambda i: (0, i)),
              pl.BlockSpec((NC, RBLK, D), lambda i: (0, i, 0)),
              pl.BlockSpec((RBLK, D), lambda i: (i, 0)),
              pl.BlockSpec((1, 1, RBLK), lambda i: (i, 0, 0)),
              pl.BlockSpec((1, D), lambda i: (0, 0)),
              pl.BlockSpec((D, D), lambda i: (0, 0)),
              pl.BlockSpec((1, D), lambda i: (0, 0))],
    out_specs=[pl.BlockSpec((N_GRAPHS, D), lambda i: (0, 0)),
               pl.BlockSpec((N_GRAPHS, D), lambda i: (0, 0))],
    out_shape=[jax.ShapeDtypeStruct((N_GRAPHS, D), jnp.float32),
               jax.ShapeDtypeStruct((N_GRAPHS, D), jnp.float32)],
    scratch_shapes=[pltpu.VMEM((N_GRAPHS, D), jnp.float32),
                    pltpu.VMEM((N_GRAPHS, D), jnp.float32)],
)


def kernel(x, edge_index, batch, W1, b1, W2, b2, Wp, bp):
    src = edge_index[0].astype(jnp.int32)
    dst = edge_index[1].astype(jnp.int32)
    epad = E_PAD - N_EDGES
    dummy = N_NODES + (jnp.arange(epad, dtype=jnp.int32) % (NPAD - N_NODES))
    src_t = jnp.concatenate([src, dummy]).reshape(NW, N_CH, CHUNK)
    dst_t = jnp.concatenate([dst, dummy]).reshape(NW, N_CH, CHUNK)

    x_pad = jnp.zeros((NPAD, D), jnp.float32).at[:N_NODES].set(x)
    batch_pad = jnp.full((NPAD,), N_GRAPHS, jnp.int32).at[:N_NODES].set(
        batch.astype(jnp.int32)).reshape(N_RBLK, 1, RBLK)

    zeros_n = jnp.zeros((NPAD,), jnp.float32)
    zrows = jnp.zeros((RPT, D), jnp.float32)
    b1r = b1.reshape(1, D)
    b2r = b2.reshape(1, D)
    wp_pad = jnp.zeros((D, D), jnp.float32).at[:, :1].set(Wp)
    bp_pad = jnp.zeros((1, D), jnp.float32).at[0, 0].set(bp[0])

    deg_parts = _deg_hist(dst_t, zeros_n)
    v1 = _mm(x_pad, W1)
    y = _scale(deg_parts, v1)
    s_parts = _edge_scatter(y, src_t, dst_t, zrows)
    z = _layer(deg_parts, s_parts, y, b1r, W2)
    t_parts = _edge_scatter(z, src_t, dst_t, zrows)
    emb, out_full = _pool(deg_parts, t_parts, z, batch_pad, b2r, wp_pad, bp_pad)
    return (out_full[:, :1], emb)
